# Initial kernel scaffold; baseline (speedup 1.0000x reference)
import jax
import jax.numpy as jnp
from jax.experimental import pallas as pl


def kernel(movies, ratings, table):
    raise NotImplementedError("write your pallas kernel here")


# speedup vs baseline: 116.1940x; 116.1940x over previous
import functools

import jax
import jax.numpy as jnp
from jax import lax
from jax.experimental import pallas as pl
from jax.experimental.pallas import tpu as pltpu
from jax.experimental.pallas import tpu_sc as plsc

R, C = 16384, 200
V, D = 2000, 9
N = R * C
L = 16
NC, NS = 2, 16
NW = NC * NS
CHUNK = N // NW
ITERS = CHUNK // L
TBL = V * D
TBL_PAD = 18048
V_PAD = 2048
RS_ITERS = V // L


def _sc_partial_sums(movies_flat, table_flat):
    mesh = plsc.VectorSubcoreMesh(core_axis_name="c", subcore_axis_name="s")

    @functools.partial(
        pl.kernel, mesh=mesh,
        out_type=jax.ShapeDtypeStruct((NW, L), jnp.float32),
        compiler_params=pltpu.CompilerParams(needs_layout_passes=False),
        scratch_types=[
            pltpu.VMEM((CHUNK,), jnp.int32),
            pltpu.VMEM((TBL_PAD,), jnp.float32),
            pltpu.VMEM((V_PAD,), jnp.float32),
            pltpu.VMEM((L,), jnp.float32),
        ],
    )
    def k(mov_hbm, tbl_hbm, out_hbm, mov_v, tbl_v, rowsum_v, acc_v):
        wid = lax.axis_index("s") * NC + lax.axis_index("c")
        pltpu.sync_copy(tbl_hbm, tbl_v.at[pl.ds(0, TBL)])
        pltpu.sync_copy(mov_hbm.at[pl.ds(wid * CHUNK, CHUNK)], mov_v)

        lane9 = lax.iota(jnp.int32, L) * D

        def rs_body(b, _):
            base = b * (L * D)
            acc = plsc.load_gather(tbl_v, [lane9 + base])
            for kk in range(1, D):
                acc = acc + plsc.load_gather(tbl_v, [lane9 + (base + kk)])
            rowsum_v[pl.ds(b * L, L)] = acc
            return 0

        lax.fori_loop(0, RS_ITERS, rs_body, 0)

        def body(i, acc):
            idx = mov_v[pl.ds(i * L, L)]
            return acc + plsc.load_gather(rowsum_v, [idx])

        acc = lax.fori_loop(0, ITERS, body, jnp.zeros((L,), jnp.float32))
        acc_v[...] = acc
        pltpu.sync_copy(acc_v, out_hbm.at[wid])

    return k(movies_flat, table_flat)


def _tc_finish(partials):
    def body(p_ref, o_ref):
        o_ref[0, 0] = jax.nn.sigmoid(jnp.sum(p_ref[...]) * (1.0 / (N * D)))

    return pl.pallas_call(
        body,
        out_shape=jax.ShapeDtypeStruct((1, 1), jnp.float32),
        out_specs=pl.BlockSpec(memory_space=pltpu.SMEM),
    )(partials)


def kernel(movies, ratings, table):
    del ratings
    partials = _sc_partial_sums(movies.reshape(-1), table.reshape(-1))
    return _tc_finish(partials)[0, 0]


# speedup vs baseline: 161.7041x; 1.3917x over previous
import functools

import jax
import jax.numpy as jnp
from jax import lax
from jax.experimental import pallas as pl
from jax.experimental.pallas import tpu as pltpu
from jax.experimental.pallas import tpu_sc as plsc

R, C = 16384, 200
V, D = 2000, 9
N = R * C
L = 16
NC, NS = 2, 16
NW = NC * NS
CHUNK = N // NW
ITERS = CHUNK // L
TBL = V * D
TBL_PAD = 18048
V_PAD = 2048
RS_ITERS = V // L


def _sc_partial_sums(movies_flat, table_flat):
    mesh = plsc.VectorSubcoreMesh(core_axis_name="c", subcore_axis_name="s")

    @functools.partial(
        pl.kernel, mesh=mesh,
        out_type=jax.ShapeDtypeStruct((NW, L), jnp.float32),
        compiler_params=pltpu.CompilerParams(needs_layout_passes=False),
        scratch_types=[
            pltpu.VMEM((CHUNK,), jnp.int32),
            pltpu.VMEM((TBL_PAD,), jnp.float32),
            pltpu.VMEM((V_PAD,), jnp.float32),
            pltpu.VMEM((L,), jnp.float32),
        ],
    )
    def k(mov_hbm, tbl_hbm, out_hbm, mov_v, tbl_v, rowsum_v, acc_v):
        wid = lax.axis_index("s") * NC + lax.axis_index("c")
        pltpu.sync_copy(tbl_hbm, tbl_v.at[pl.ds(0, TBL)])
        pltpu.sync_copy(mov_hbm.at[pl.ds(wid * CHUNK, CHUNK)], mov_v)

        lane9 = lax.iota(jnp.int32, L) * D

        def rs_body(b, _):
            base = b * (L * D)
            acc = plsc.load_gather(tbl_v, [lane9 + base])
            for kk in range(1, D):
                acc = acc + plsc.load_gather(tbl_v, [lane9 + (base + kk)])
            rowsum_v[pl.ds(b * L, L)] = acc
            return 0

        lax.fori_loop(0, RS_ITERS, rs_body, 0)

        U = 8

        def body(i, accs):
            accs = list(accs)
            base = i * (U * L)
            for u in range(U):
                idx = mov_v[pl.ds(base + u * L, L)]
                g = plsc.load_gather(rowsum_v, [idx])
                accs[u % 4] = accs[u % 4] + g
            return tuple(accs)

        zero = jnp.zeros((L,), jnp.float32)
        a0, a1, a2, a3 = lax.fori_loop(
            0, ITERS // U, body, (zero, zero, zero, zero))
        acc_v[...] = (a0 + a1) + (a2 + a3)
        pltpu.sync_copy(acc_v, out_hbm.at[wid])

    return k(movies_flat, table_flat)


def _tc_finish(partials):
    def body(p_ref, o_ref):
        o_ref[0, 0] = jax.nn.sigmoid(jnp.sum(p_ref[...]) * (1.0 / (N * D)))

    return pl.pallas_call(
        body,
        out_shape=jax.ShapeDtypeStruct((1, 1), jnp.float32),
        out_specs=pl.BlockSpec(memory_space=pltpu.SMEM),
    )(partials)


def kernel(movies, ratings, table):
    del ratings
    partials = _sc_partial_sums(movies.reshape(-1), table.reshape(-1))
    return _tc_finish(partials)[0, 0]


# speedup vs baseline: 166.3669x; 1.0288x over previous
import functools

import jax
import jax.numpy as jnp
from jax import lax
from jax.experimental import pallas as pl
from jax.experimental.pallas import tpu as pltpu
from jax.experimental.pallas import tpu_sc as plsc

R, C = 16384, 200
V, D = 2000, 9
N = R * C
L = 16
NC, NS = 2, 16
NW = NC * NS
ROWS = R // NW
SLAB = ROWS // 2
KFULL = C // L
TAIL = C - KFULL * L
V_PAD = 2048
RS_ITERS = V // L


def _sc_partial_sums(movies, table):
    mesh = plsc.VectorSubcoreMesh(core_axis_name="c", subcore_axis_name="s")

    @functools.partial(
        pl.kernel, mesh=mesh,
        out_type=jax.ShapeDtypeStruct((NW, L), jnp.float32),
        compiler_params=pltpu.CompilerParams(
            needs_layout_passes=False, use_tc_tiling_on_sc=False),
        scratch_types=[
            pltpu.VMEM((SLAB, C), jnp.int32),
            pltpu.VMEM((SLAB, C), jnp.int32),
            pltpu.VMEM((V * D + 48,), jnp.float32),
            pltpu.VMEM((V_PAD,), jnp.float32),
            pltpu.VMEM((L,), jnp.float32),
            pltpu.SemaphoreType.DMA,
            pltpu.SemaphoreType.DMA,
        ],
    )
    def k(mov_hbm, tbl_hbm, out_hbm, mov_a, mov_b, tbl_v, rowsum_v, acc_v,
          sem_a, sem_b):
        wid = lax.axis_index("s") * NC + lax.axis_index("c")
        base = wid * ROWS
        pltpu.sync_copy(tbl_hbm, tbl_v.at[pl.ds(0, V * D)])
        h_a = pltpu.async_copy(mov_hbm.at[pl.ds(base, SLAB)], mov_a, sem_a)

        lane = lax.iota(jnp.int32, L)

        lane9 = lane * D

        def rs_body(b, _):
            flat_base = b * (L * D)
            acc = plsc.load_gather(tbl_v, [lane9 + flat_base])
            for kk in range(1, D):
                acc = acc + plsc.load_gather(tbl_v, [lane9 + (flat_base + kk)])
            rowsum_v[pl.ds(b * L, L)] = acc
            return 0

        lax.fori_loop(0, RS_ITERS, rs_body, 0)
        rowsum_v[pl.ds(V, L)] = jnp.zeros((L,), jnp.float32)

        h_a.wait()
        h_b = pltpu.async_copy(
            mov_hbm.at[pl.ds(base + SLAB, SLAB)], mov_b, sem_b)

        tail_mask = lane < (L - TAIL)
        null_idx = jnp.full((L,), V, jnp.int32)

        def make_body(mov_v):
            def body(r, accs):
                accs = list(accs)
                for kk in range(KFULL):
                    idx = mov_v[r, pl.ds(kk * L, L)]
                    g = plsc.load_gather(rowsum_v, [idx])
                    accs[kk % 4] = accs[kk % 4] + g
                idx = mov_v[r, pl.ds(C - L, L)]
                idx = jnp.where(tail_mask, null_idx, idx)
                g = plsc.load_gather(rowsum_v, [idx])
                accs[KFULL % 4] = accs[KFULL % 4] + g
                return tuple(accs)
            return body

        zero = jnp.zeros((L,), jnp.float32)
        accs = lax.fori_loop(0, SLAB, make_body(mov_a), (zero,) * 4)
        h_b.wait()
        a0, a1, a2, a3 = lax.fori_loop(0, SLAB, make_body(mov_b), accs)
        acc_v[...] = (a0 + a1) + (a2 + a3)
        pltpu.sync_copy(acc_v, out_hbm.at[wid])

    return k(movies, table)


def _tc_finish(partials):
    def body(p_ref, o_ref):
        o_ref[0, 0] = jax.nn.sigmoid(jnp.sum(p_ref[...]) * (1.0 / (N * D)))

    return pl.pallas_call(
        body,
        out_shape=jax.ShapeDtypeStruct((1, 1), jnp.float32),
        out_specs=pl.BlockSpec(memory_space=pltpu.SMEM),
    )(partials)


def kernel(movies, ratings, table):
    del ratings
    partials = _sc_partial_sums(movies, table.reshape(-1))
    return _tc_finish(partials)[0, 0]


# speedup vs baseline: 354.5572x; 2.1312x over previous
import functools

import jax
import jax.numpy as jnp
from jax import lax
from jax.experimental import pallas as pl
from jax.experimental.pallas import tpu as pltpu
from jax.experimental.pallas import tpu_sc as plsc

R, C = 16384, 200
V, D = 2000, 9
N = R * C
L = 16
NC, NS = 2, 16
NW = NC * NS
COLS = R // NW
SLAB = COLS // 2
KS = SLAB // L
TBL_PAD = 18048
V_PAD = 2048
RS_ITERS = V // L


def _sc_partial_sums(movies_t, table_flat):
    mesh = plsc.VectorSubcoreMesh(core_axis_name="c", subcore_axis_name="s")

    @functools.partial(
        pl.kernel, mesh=mesh,
        out_type=jax.ShapeDtypeStruct((NW, L), jnp.float32),
        compiler_params=pltpu.CompilerParams(needs_layout_passes=False),
        scratch_types=[
            pltpu.VMEM((C, SLAB), jnp.int32),
            pltpu.VMEM((C, SLAB), jnp.int32),
            pltpu.VMEM((TBL_PAD,), jnp.float32),
            pltpu.VMEM((V_PAD,), jnp.float32),
            pltpu.VMEM((L,), jnp.float32),
            pltpu.SemaphoreType.DMA,
            pltpu.SemaphoreType.DMA,
        ],
    )
    def k(mov_hbm, tbl_hbm, out_hbm, mov_a, mov_b, tbl_v, rowsum_v, acc_v,
          sem_a, sem_b):
        wid = lax.axis_index("s") * NC + lax.axis_index("c")
        base = wid * COLS
        pltpu.sync_copy(tbl_hbm, tbl_v.at[pl.ds(0, V * D)])
        h_a = pltpu.async_copy(
            mov_hbm.at[:, pl.ds(base, SLAB)], mov_a, sem_a)

        lane9 = lax.iota(jnp.int32, L) * D

        def rs_body(b, _):
            flat_base = b * (L * D)
            acc = plsc.load_gather(tbl_v, [lane9 + flat_base])
            for kk in range(1, D):
                acc = acc + plsc.load_gather(tbl_v, [lane9 + (flat_base + kk)])
            rowsum_v[pl.ds(b * L, L)] = acc
            return 0

        lax.fori_loop(0, RS_ITERS, rs_body, 0)

        h_a.wait()
        h_b = pltpu.async_copy(
            mov_hbm.at[:, pl.ds(base + SLAB, SLAB)], mov_b, sem_b)

        def make_body(mov_v):
            def body(r, accs):
                accs = list(accs)
                for kk in range(KS):
                    idx = mov_v[r, pl.ds(kk * L, L)]
                    g = plsc.load_gather(rowsum_v, [idx])
                    accs[kk % 4] = accs[kk % 4] + g
                return tuple(accs)
            return body

        zero = jnp.zeros((L,), jnp.float32)
        accs = lax.fori_loop(0, C, make_body(mov_a), (zero,) * 4)
        h_b.wait()
        a0, a1, a2, a3 = lax.fori_loop(0, C, make_body(mov_b), accs)
        acc_v[...] = (a0 + a1) + (a2 + a3)
        pltpu.sync_copy(acc_v, out_hbm.at[wid])

    return k(movies_t, table_flat)


def _tc_finish(partials):
    def body(p_ref, o_ref):
        o_ref[0, 0] = jax.nn.sigmoid(jnp.sum(p_ref[...]) * (1.0 / (N * D)))

    return pl.pallas_call(
        body,
        out_shape=jax.ShapeDtypeStruct((1, 1), jnp.float32),
        out_specs=pl.BlockSpec(memory_space=pltpu.SMEM),
    )(partials)


def kernel(movies, ratings, table):
    del ratings
    partials = _sc_partial_sums(movies.T, table.reshape(-1))
    return _tc_finish(partials)[0, 0]
